# Initial kernel scaffold; baseline (speedup 1.0000x reference)
import jax
import jax.numpy as jnp
from jax.experimental import pallas as pl


def kernel(cell_indices, neighbor_indices, states, local_distance_threshold, functional_distance_threshold, distant_distance_threshold, functional_similarity_threshold):
    raise NotImplementedError("write your pallas kernel here")


# speedup vs baseline: 9.0970x; 9.0970x over previous
import functools

import jax
import jax.numpy as jnp
import numpy as np
from jax import lax
from jax.experimental import pallas as pl
from jax.experimental.pallas import tpu as pltpu
from jax.experimental.pallas import tpu_sc as plsc

LX, LY, LZ = 50, 50, 40
N_CELLS = LX * LY * LZ
STATE_SIZE = 32
BATCH = 32768
MAX_NEIGHBORS = 26

NC, NS, LANES = 2, 16, 16
NW = NC * NS
BATCH_PER_W = BATCH // NW
CHUNK_B = 64
N_CHUNKS = BATCH_PER_W // CHUNK_B
P_CHUNK = CHUNK_B * MAX_NEIGHBORS
GROUPS = P_CHUNK // LANES
NB_GATHERS = P_CHUNK // 128
PAIRS = BATCH * MAX_NEIGHBORS

_INV_LXY = 1.0 / (LX * LY)
_INV_LX = 1.0 / LX

_BOFP = np.arange(P_CHUNK, dtype=np.int32) // MAX_NEIGHBORS


def _normalize_body(x_ref, o_ref):
    x = x_ref[...]
    n2 = jnp.sum(x * x, axis=1, keepdims=True)
    o_ref[...] = x * (1.0 / jnp.sqrt(n2 + 1e-12))


def _normalize_states(states):
    rows = states.shape[0]
    blk = 2000
    return pl.pallas_call(
        _normalize_body,
        grid=(rows // blk,),
        in_specs=[pl.BlockSpec((blk, STATE_SIZE), lambda i: (i, 0))],
        out_specs=pl.BlockSpec((blk, STATE_SIZE), lambda i: (i, 0)),
        out_shape=jax.ShapeDtypeStruct((rows, STATE_SIZE), jnp.float32),
    )(states)


def _classify_body(cell2d, nb2d, states_h, thr_h, bofp_h, out_h,
                   cellidx_v, nbidx2_v, crows_v, nrows_v,
                   bofp_v, thr_v, out4_v, sem):
    cid = lax.axis_index("c")
    sid = lax.axis_index("s")
    wid = sid * NC + cid

    pltpu.sync_copy(thr_h, thr_v)
    pltpu.sync_copy(bofp_h, bofp_v)
    tv = thr_v[...]
    lt2v = jnp.full((LANES,), tv[0], jnp.float32)
    ft2v = jnp.full((LANES,), tv[1], jnp.float32)
    stv = jnp.full((LANES,), tv[2], jnp.float32)
    onev = jnp.full((LANES,), 1.0, jnp.float32)
    zerov = jnp.full((LANES,), 0.0, jnp.float32)
    lanes = lax.iota(jnp.int32, LANES)

    def decode(idx):
        f = idx.astype(jnp.float32)
        z = ((f + 0.5) * _INV_LXY).astype(jnp.int32)
        rem = idx - z * (LX * LY)
        y = ((rem.astype(jnp.float32) + 0.5) * _INV_LX).astype(jnp.int32)
        x = rem - y * LX
        return x, y, z

    def chunk_body(ci, carry):
        row = wid * N_CHUNKS + ci
        poff = (wid * BATCH_PER_W + ci * CHUNK_B) * MAX_NEIGHBORS

        pltpu.sync_copy(cell2d.at[row], cellidx_v)
        pltpu.sync_copy(nb2d.at[row], nbidx2_v)

        descs = [pltpu.async_copy(states_h.at[cellidx_v], crows_v, sem)]
        for g in range(NB_GATHERS):
            descs.append(pltpu.async_copy(
                states_h.at[nbidx2_v.at[g]],
                nrows_v.at[pl.ds(g * 128, 128)], sem))
        for d in descs:
            d.wait()

        def group_body(g, carry2):
            base = g * LANES
            pvec = base + lanes
            grow = lax.shift_right_logical(g, 3)
            gcol = lax.shift_left(lax.rem(g, 8), 4)
            nidx = nbidx2_v[grow, pl.ds(gcol, LANES)]
            bvec = bofp_v[pl.ds(base, LANES)]
            cidx = plsc.load_gather(cellidx_v, [bvec])

            cx, cy, cz = decode(cidx)
            nx, ny, nz = decode(nidx)
            dx = cx - nx
            dy = cy - ny
            dz = cz - nz
            d2 = (dx * dx + dy * dy + dz * dz).astype(jnp.float32)

            acc = zerov
            for k in range(STATE_SIZE):
                kv = jnp.full((LANES,), k, jnp.int32)
                nvals = plsc.load_gather(nrows_v, [pvec, kv])
                cvals = plsc.load_gather(crows_v, [bvec, kv])
                acc = acc + nvals * cvals

            valid = nidx >= 0
            local_m = valid & (d2 <= lt2v)
            func_m = valid & (d2 > lt2v) & (d2 <= ft2v) & (acc >= stv)
            dist_m = valid & (~local_m) & (~func_m)

            out4_v[0, pl.ds(base, LANES)] = jnp.where(local_m, onev, zerov)
            out4_v[1, pl.ds(base, LANES)] = jnp.where(func_m, onev, zerov)
            out4_v[2, pl.ds(base, LANES)] = jnp.where(dist_m, onev, zerov)
            out4_v[3, pl.ds(base, LANES)] = jnp.where(valid, acc, zerov)
            return carry2

        lax.fori_loop(0, GROUPS, group_body, 0)

        for p in range(4):
            pltpu.sync_copy(out4_v.at[p], out_h.at[p, pl.ds(poff, P_CHUNK)])
        return carry

    lax.fori_loop(0, N_CHUNKS, chunk_body, 0)


@functools.partial(
    pl.kernel,
    out_type=jax.ShapeDtypeStruct((4, PAIRS), jnp.float32),
    mesh=plsc.VectorSubcoreMesh(core_axis_name="c", subcore_axis_name="s",
                                num_cores=NC, num_subcores=NS),
    scratch_types=[
        pltpu.VMEM((CHUNK_B,), jnp.int32),
        pltpu.VMEM((NB_GATHERS, 128), jnp.int32),
        pltpu.VMEM((CHUNK_B, STATE_SIZE), jnp.float32),
        pltpu.VMEM((P_CHUNK, STATE_SIZE), jnp.float32),
        pltpu.VMEM((P_CHUNK,), jnp.int32),
        pltpu.VMEM((16,), jnp.float32),
        pltpu.VMEM((4, P_CHUNK), jnp.float32),
        pltpu.SemaphoreType.DMA,
    ],
    compiler_params=pltpu.CompilerParams(needs_layout_passes=False,
                                         use_tc_tiling_on_sc=False),
)
def _classify(cell2d, nb2d, states_h, thr_h, bofp_h, out_h, *scratch):
    _classify_body(cell2d, nb2d, states_h, thr_h, bofp_h, out_h, *scratch)


def kernel(cell_indices, neighbor_indices, states,
           local_distance_threshold, functional_distance_threshold,
           distant_distance_threshold, functional_similarity_threshold):
    del distant_distance_threshold
    states_n = _normalize_states(states)

    cell2d = cell_indices.reshape(NW * N_CHUNKS, CHUNK_B)
    nb2d = neighbor_indices.reshape(NW * N_CHUNKS, NB_GATHERS, 128)
    thr = jnp.concatenate([
        jnp.stack([local_distance_threshold * local_distance_threshold,
                   functional_distance_threshold * functional_distance_threshold,
                   functional_similarity_threshold]).astype(jnp.float32),
        jnp.zeros((13,), jnp.float32),
    ])
    bofp = jnp.asarray(_BOFP)

    out = _classify(cell2d, nb2d, states_n, thr, bofp)
    return out.reshape(4, BATCH, MAX_NEIGHBORS)


# speedup vs baseline: 12.0763x; 1.3275x over previous
import functools

import jax
import jax.numpy as jnp
import numpy as np
from jax import lax
from jax.experimental import pallas as pl
from jax.experimental.pallas import tpu as pltpu
from jax.experimental.pallas import tpu_sc as plsc

LX, LY, LZ = 50, 50, 40
N_CELLS = LX * LY * LZ
STATE_SIZE = 32
BATCH = 32768
MAX_NEIGHBORS = 26

NC, NS, LANES = 2, 16, 16
NW = NC * NS
BATCH_PER_W = BATCH // NW
CHUNK_B = 64
N_CHUNKS = BATCH_PER_W // CHUNK_B
P_CHUNK = CHUNK_B * MAX_NEIGHBORS
GROUPS = P_CHUNK // LANES
NB_GATHERS = P_CHUNK // 128
ROW_PITCH = 33
PAIRS = BATCH * MAX_NEIGHBORS

_INV_LXY = 1.0 / (LX * LY)
_INV_LX = 1.0 / LX

_BOFP = np.arange(P_CHUNK, dtype=np.int32) // MAX_NEIGHBORS


def _normalize_body(x_ref, o_ref):
    x = x_ref[...]
    n2 = jnp.sum(x * x, axis=1, keepdims=True)
    o_ref[...] = x * (1.0 / jnp.sqrt(n2 + 1e-12))


def _normalize_states(states):
    rows = states.shape[0]
    blk = 2000
    return pl.pallas_call(
        _normalize_body,
        grid=(rows // blk,),
        in_specs=[pl.BlockSpec((blk, STATE_SIZE), lambda i: (i, 0))],
        out_specs=pl.BlockSpec((blk, STATE_SIZE), lambda i: (i, 0)),
        out_shape=jax.ShapeDtypeStruct((rows, STATE_SIZE), jnp.float32),
    )(states)


def _classify_body(cell2d, nb2d, states_h, thr_h, bofp_h, out_h,
                   cellidx_v, nbidx2_v, crows_v, nrows_v,
                   bofp_v, thr_v, out4_v, sim_v, sem):
    cid = lax.axis_index("c")
    sid = lax.axis_index("s")
    wid = sid * NC + cid

    pltpu.sync_copy(thr_h, thr_v)
    pltpu.sync_copy(bofp_h, bofp_v)
    tv = thr_v[...]
    lt2v = jnp.full((LANES,), tv[0], jnp.float32)
    ft2v = jnp.full((LANES,), tv[1], jnp.float32)
    stv = jnp.full((LANES,), tv[2], jnp.float32)
    onev = jnp.full((LANES,), 1.0, jnp.float32)
    zerov = jnp.full((LANES,), 0.0, jnp.float32)
    lanes = lax.iota(jnp.int32, LANES)
    last_lane = lanes == (LANES - 1)

    def decode(idx):
        f = idx.astype(jnp.float32)
        z = ((f + 0.5) * _INV_LXY).astype(jnp.int32)
        rem = idx - z * (LX * LY)
        y = ((rem.astype(jnp.float32) + 0.5) * _INV_LX).astype(jnp.int32)
        x = rem - y * LX
        return x, y, z

    def chunk_body(ci, carry):
        row = wid * N_CHUNKS + ci
        poff = (wid * BATCH_PER_W + ci * CHUNK_B) * MAX_NEIGHBORS

        pltpu.sync_copy(cell2d.at[row], cellidx_v)
        pltpu.sync_copy(nb2d.at[row], nbidx2_v)

        descs = [pltpu.async_copy(states_h.at[cellidx_v], crows_v, sem)]
        for g in range(NB_GATHERS):
            descs.append(pltpu.async_copy(
                states_h.at[nbidx2_v.at[g]],
                nrows_v.at[pl.ds(g * 128, 128)], sem))
        for d in descs:
            d.wait()

        def dot_body(b, carry2):
            c0 = crows_v[b, pl.ds(0, LANES)]
            c1 = crows_v[b, pl.ds(LANES, LANES)]
            p0 = b * MAX_NEIGHBORS
            for j in range(MAX_NEIGHBORS):
                n0 = nrows_v[p0 + j, pl.ds(0, LANES)]
                n1 = nrows_v[p0 + j, pl.ds(LANES, LANES)]
                s = n0 * c0 + n1 * c1
                cs = plsc.cumsum(s)
                pidx = jnp.full((LANES,), p0 + j, jnp.int32)
                plsc.store_scatter(sim_v, [pidx], cs, mask=last_lane)
            return carry2

        lax.fori_loop(0, CHUNK_B, dot_body, 0)

        def group_body(g, carry2):
            base = g * LANES
            grow = lax.shift_right_logical(g, 3)
            gcol = lax.shift_left(lax.rem(g, 8), 4)
            nidx = nbidx2_v[grow, pl.ds(gcol, LANES)]
            bvec = bofp_v[pl.ds(base, LANES)]
            cidx = plsc.load_gather(cellidx_v, [bvec])
            simv = sim_v[pl.ds(base, LANES)]

            cx, cy, cz = decode(cidx)
            nx, ny, nz = decode(nidx)
            dx = cx - nx
            dy = cy - ny
            dz = cz - nz
            d2 = (dx * dx + dy * dy + dz * dz).astype(jnp.float32)

            valid = nidx >= 0
            local_m = valid & (d2 <= lt2v)
            func_m = valid & (d2 > lt2v) & (d2 <= ft2v) & (simv >= stv)
            dist_m = valid & (~local_m) & (~func_m)

            out4_v[0, pl.ds(base, LANES)] = jnp.where(local_m, onev, zerov)
            out4_v[1, pl.ds(base, LANES)] = jnp.where(func_m, onev, zerov)
            out4_v[2, pl.ds(base, LANES)] = jnp.where(dist_m, onev, zerov)
            out4_v[3, pl.ds(base, LANES)] = jnp.where(valid, simv, zerov)
            return carry2

        lax.fori_loop(0, GROUPS, group_body, 0)

        for p in range(4):
            pltpu.sync_copy(out4_v.at[p], out_h.at[p, pl.ds(poff, P_CHUNK)])
        return carry

    lax.fori_loop(0, N_CHUNKS, chunk_body, 0)


@functools.partial(
    pl.kernel,
    out_type=jax.ShapeDtypeStruct((4, PAIRS), jnp.float32),
    mesh=plsc.VectorSubcoreMesh(core_axis_name="c", subcore_axis_name="s",
                                num_cores=NC, num_subcores=NS),
    scratch_types=[
        pltpu.VMEM((CHUNK_B,), jnp.int32),
        pltpu.VMEM((NB_GATHERS, 128), jnp.int32),
        pltpu.VMEM((CHUNK_B, STATE_SIZE), jnp.float32),
        pltpu.VMEM((P_CHUNK, STATE_SIZE), jnp.float32),
        pltpu.VMEM((P_CHUNK,), jnp.int32),
        pltpu.VMEM((16,), jnp.float32),
        pltpu.VMEM((4, P_CHUNK), jnp.float32),
        pltpu.VMEM((P_CHUNK,), jnp.float32),
        pltpu.SemaphoreType.DMA,
    ],
    compiler_params=pltpu.CompilerParams(needs_layout_passes=False,
                                         use_tc_tiling_on_sc=False),
)
def _classify(cell2d, nb2d, states_h, thr_h, bofp_h, out_h, *scratch):
    _classify_body(cell2d, nb2d, states_h, thr_h, bofp_h, out_h, *scratch)


def kernel(cell_indices, neighbor_indices, states,
           local_distance_threshold, functional_distance_threshold,
           distant_distance_threshold, functional_similarity_threshold):
    del distant_distance_threshold
    states_n = _normalize_states(states)

    cell2d = cell_indices.reshape(NW * N_CHUNKS, CHUNK_B)
    nb2d = neighbor_indices.reshape(NW * N_CHUNKS, NB_GATHERS, 128)
    thr = jnp.concatenate([
        jnp.stack([local_distance_threshold * local_distance_threshold,
                   functional_distance_threshold * functional_distance_threshold,
                   functional_similarity_threshold]).astype(jnp.float32),
        jnp.zeros((13,), jnp.float32),
    ])
    bofp = jnp.asarray(_BOFP)

    out = _classify(cell2d, nb2d, states_n, thr, bofp)
    return out.reshape(4, BATCH, MAX_NEIGHBORS)


# speedup vs baseline: 19.5449x; 1.6185x over previous
import functools

import jax
import jax.numpy as jnp
import numpy as np
from jax import lax
from jax.experimental import pallas as pl
from jax.experimental.pallas import tpu as pltpu
from jax.experimental.pallas import tpu_sc as plsc

LX, LY, LZ = 50, 50, 40
N_CELLS = LX * LY * LZ
STATE_SIZE = 32
BATCH = 32768
MAX_NEIGHBORS = 26

NC, NS, LANES = 2, 16, 16
NW = NC * NS
BATCH_PER_W = BATCH // NW
CHUNK_B = 64
N_CHUNKS = BATCH_PER_W // CHUNK_B
P_CHUNK = CHUNK_B * MAX_NEIGHBORS
GROUPS = P_CHUNK // LANES
NB_GATHERS = P_CHUNK // 128
ROW_PITCH = 33
PAIRS = BATCH * MAX_NEIGHBORS

_INV_LXY = 1.0 / (LX * LY)
_INV_LX = 1.0 / LX


def _normalize_body(x_ref, o_ref):
    x = x_ref[...]
    n2 = jnp.sum(x * x, axis=1, keepdims=True)
    o_ref[...] = x * (1.0 / jnp.sqrt(n2 + 1e-12))


def _normalize_states(states):
    rows = states.shape[0]
    blk = 2000
    return pl.pallas_call(
        _normalize_body,
        grid=(rows // blk,),
        in_specs=[pl.BlockSpec((blk, STATE_SIZE), lambda i: (i, 0))],
        out_specs=pl.BlockSpec((blk, STATE_SIZE), lambda i: (i, 0)),
        out_shape=jax.ShapeDtypeStruct((rows, STATE_SIZE), jnp.float32),
    )(states)


def _classify_body(cell2d, nb2d, states_h, thr_h, out_h,
                   cellidx_v, nbidx2_v, crows_v, nrows_v,
                   thr_v, out4_v, sem):
    cid = lax.axis_index("c")
    sid = lax.axis_index("s")
    wid = sid * NC + cid

    pltpu.sync_copy(thr_h, thr_v)
    tv = thr_v[...]
    lt2v = jnp.full((LANES,), tv[0], jnp.float32)
    ft2v = jnp.full((LANES,), tv[1], jnp.float32)
    stv = jnp.full((LANES,), tv[2], jnp.float32)
    onev = jnp.full((LANES,), 1.0, jnp.float32)
    zerov = jnp.full((LANES,), 0.0, jnp.float32)
    lanes = lax.iota(jnp.int32, LANES)
    shfl_idx = [lax.bitwise_xor(lanes, k) for k in (1, 2, 4, 8)]
    shfl_msk = [(lax.bitwise_and(lanes, k) != 0) for k in (1, 2, 4, 8)]

    gdn = lax.GatherDimensionNumbers(offset_dims=(), collapsed_slice_dims=(0,),
                                     start_index_map=(0,))

    def shfl(v, idxv):
        return lax.gather(v, idxv[:, None], dimension_numbers=gdn,
                          slice_sizes=(1,),
                          mode=lax.GatherScatterMode.PROMISE_IN_BOUNDS)

    def reduce16(svecs):
        lvl = svecs
        for idxv, mk in zip(shfl_idx, shfl_msk):
            nxt = []
            for i in range(0, len(lvl), 2):
                sa = lvl[i] + shfl(lvl[i], idxv)
                sb = lvl[i + 1] + shfl(lvl[i + 1], idxv)
                nxt.append(jnp.where(mk, sb, sa))
            lvl = nxt
        return lvl[0]

    def decode(idx):
        f = idx.astype(jnp.float32)
        z = ((f + 0.5) * _INV_LXY).astype(jnp.int32)
        rem = idx - z * (LX * LY)
        y = ((rem.astype(jnp.float32) + 0.5) * _INV_LX).astype(jnp.int32)
        x = rem - y * LX
        return x, y, z

    def chunk_body(ci, carry):
        row = wid * N_CHUNKS + ci
        poff = (wid * BATCH_PER_W + ci * CHUNK_B) * MAX_NEIGHBORS

        pltpu.sync_copy(cell2d.at[row], cellidx_v)
        pltpu.sync_copy(nb2d.at[row], nbidx2_v)

        descs = [pltpu.async_copy(states_h.at[cellidx_v], crows_v, sem)]
        for g in range(NB_GATHERS):
            descs.append(pltpu.async_copy(
                states_h.at[nbidx2_v.at[g]],
                nrows_v.at[pl.ds(g * 128, 128)], sem))
        for d in descs:
            d.wait()

        def group_body(g, carry2):
            b0, r0 = carry2
            base = g * LANES
            svecs = []
            for j in range(LANES):
                p = base + j
                bj = b0 + (r0 + j >= MAX_NEIGHBORS).astype(jnp.int32)
                n0 = nrows_v[p, pl.ds(0, LANES)]
                n1 = nrows_v[p, pl.ds(LANES, LANES)]
                c0 = crows_v[bj, pl.ds(0, LANES)]
                c1 = crows_v[bj, pl.ds(LANES, LANES)]
                svecs.append(n0 * c0 + n1 * c1)
            simv = reduce16(svecs)

            grow = lax.shift_right_logical(g, 3)
            gcol = lax.shift_left(lax.rem(g, 8), 4)
            nidx = nbidx2_v[grow, pl.ds(gcol, LANES)]
            bvec = b0 + (r0 + lanes >= MAX_NEIGHBORS).astype(jnp.int32)
            cidx = plsc.load_gather(cellidx_v, [bvec])

            cx, cy, cz = decode(cidx)
            nx, ny, nz = decode(nidx)
            dx = cx - nx
            dy = cy - ny
            dz = cz - nz
            d2 = (dx * dx + dy * dy + dz * dz).astype(jnp.float32)

            valid = nidx >= 0
            local_m = valid & (d2 <= lt2v)
            func_m = valid & (d2 > lt2v) & (d2 <= ft2v) & (simv >= stv)
            dist_m = valid & (~local_m) & (~func_m)

            out4_v[0, pl.ds(base, LANES)] = jnp.where(local_m, onev, zerov)
            out4_v[1, pl.ds(base, LANES)] = jnp.where(func_m, onev, zerov)
            out4_v[2, pl.ds(base, LANES)] = jnp.where(dist_m, onev, zerov)
            out4_v[3, pl.ds(base, LANES)] = jnp.where(valid, simv, zerov)

            r1 = r0 + LANES
            wrap = (r1 >= MAX_NEIGHBORS).astype(jnp.int32)
            return (b0 + wrap, r1 - wrap * MAX_NEIGHBORS)

        lax.fori_loop(0, GROUPS, group_body,
                      (jnp.int32(0), jnp.int32(0)))

        for p in range(4):
            pltpu.sync_copy(out4_v.at[p], out_h.at[p, pl.ds(poff, P_CHUNK)])
        return carry

    lax.fori_loop(0, N_CHUNKS, chunk_body, 0)


@functools.partial(
    pl.kernel,
    out_type=jax.ShapeDtypeStruct((4, PAIRS), jnp.float32),
    mesh=plsc.VectorSubcoreMesh(core_axis_name="c", subcore_axis_name="s",
                                num_cores=NC, num_subcores=NS),
    scratch_types=[
        pltpu.VMEM((CHUNK_B,), jnp.int32),
        pltpu.VMEM((NB_GATHERS, 128), jnp.int32),
        pltpu.VMEM((CHUNK_B, STATE_SIZE), jnp.float32),
        pltpu.VMEM((P_CHUNK, STATE_SIZE), jnp.float32),
        pltpu.VMEM((16,), jnp.float32),
        pltpu.VMEM((4, P_CHUNK), jnp.float32),
        pltpu.SemaphoreType.DMA,
    ],
    compiler_params=pltpu.CompilerParams(needs_layout_passes=False,
                                         use_tc_tiling_on_sc=False),
)
def _classify(cell2d, nb2d, states_h, thr_h, out_h, *scratch):
    _classify_body(cell2d, nb2d, states_h, thr_h, out_h, *scratch)


def kernel(cell_indices, neighbor_indices, states,
           local_distance_threshold, functional_distance_threshold,
           distant_distance_threshold, functional_similarity_threshold):
    del distant_distance_threshold
    states_n = _normalize_states(states)

    cell2d = cell_indices.reshape(NW * N_CHUNKS, CHUNK_B)
    nb2d = neighbor_indices.reshape(NW * N_CHUNKS, NB_GATHERS, 128)
    thr = jnp.concatenate([
        jnp.stack([local_distance_threshold * local_distance_threshold,
                   functional_distance_threshold * functional_distance_threshold,
                   functional_similarity_threshold]).astype(jnp.float32),
        jnp.zeros((13,), jnp.float32),
    ])
    out = _classify(cell2d, nb2d, states_n, thr)
    return out.reshape(4, BATCH, MAX_NEIGHBORS)
